# Initial kernel scaffold; baseline (speedup 1.0000x reference)
import jax
import jax.numpy as jnp
from jax.experimental import pallas as pl


def kernel(obs, acs, cost_matrix):
    raise NotImplementedError("write your pallas kernel here")


# speedup vs baseline: 4.4401x; 4.4401x over previous
import functools

import jax
import jax.numpy as jnp
from jax import lax
from jax.experimental import pallas as pl
from jax.experimental.pallas import tpu as pltpu
from jax.experimental.pallas import tpu_sc as plsc

H = 1024
W = 1024
A = 8
N = 1048576

NC = 2
NS = 16
NW = NC * NS

CH = N // NW
SUB = 4096
ROWS = SUB // 128

P = 4
BINS = H * W * A
SLICE = BINS // (NC * P)
C = SLICE // A
TRASH = 2048
PERS = N // NS
CHB = 8192
CROWS = CHB // 128
CELLS_T = C // NS
OBC = 2048
ZB = 8192

_mesh = plsc.VectorSubcoreMesh(core_axis_name="c", subcore_axis_name="s")


@functools.partial(
    pl.kernel,
    out_type=(
        jax.ShapeDtypeStruct((N,), jnp.float32),
        jax.ShapeDtypeStruct((N,), jnp.int32),
    ),
    mesh=_mesh,
    scratch_types=[
        pltpu.VMEM((SUB,), jnp.int32),
        pltpu.VMEM((SUB,), jnp.int32),
        pltpu.VMEM((SUB,), jnp.int32),
        pltpu.VMEM((ROWS, 128), jnp.int32),
        pltpu.VMEM((SUB,), jnp.int32),
        pltpu.VMEM((SUB,), jnp.float32),
        pltpu.VMEM_SHARED((H * W,), jnp.float32),
        pltpu.SemaphoreType.DMA,
    ],
)
def _cost_bins(obs_h, obs_w, acs, table, cost_out, bins_out,
               hbuf, wbuf, abuf, cellrows, binbuf, costbuf, table_sp, sem):
    c = lax.axis_index("c")
    s = lax.axis_index("s")
    wid = s * NC + c
    tchunk = (H * W) // NS
    pltpu.sync_copy(table.at[pl.ds(s * tchunk, tchunk)],
                    table_sp.at[pl.ds(s * tchunk, tchunk)])
    plsc.subcore_barrier()
    base = wid * CH

    def chunk_body(j, carry):
        off = base + j * SUB
        pltpu.sync_copy(obs_h.at[pl.ds(off, SUB)], hbuf)
        pltpu.sync_copy(obs_w.at[pl.ds(off, SUB)], wbuf)
        pltpu.sync_copy(acs.at[pl.ds(off, SUB)], abuf)

        def row_body(r, carry2):
            def vec_body(q, carry3):
                o = r * 128 + q * 16
                hh = hbuf[pl.ds(o, 16)]
                ww = wbuf[pl.ds(o, 16)]
                aa = abuf[pl.ds(o, 16)]
                cell = hh * W + ww
                cellrows[r, pl.ds(q * 16, 16)] = cell
                binbuf[pl.ds(o, 16)] = cell * A + aa
                return carry3
            return lax.fori_loop(0, 8, vec_body, carry2)
        lax.fori_loop(0, ROWS, row_body, 0)

        def fire(r, carry2):
            pltpu.async_copy(table_sp.at[cellrows.at[r]],
                             costbuf.at[pl.ds(r * 128, 128)], sem)
            return carry2
        lax.fori_loop(0, ROWS, fire, 0)

        def drain(r, carry2):
            pltpu.make_async_copy(table_sp.at[cellrows.at[0]],
                                  costbuf.at[pl.ds(0, 128)], sem).wait()
            return carry2
        lax.fori_loop(0, ROWS, drain, 0)

        pltpu.sync_copy(costbuf, cost_out.at[pl.ds(off, SUB)])
        pltpu.sync_copy(binbuf, bins_out.at[pl.ds(off, SUB)])
        return carry
    lax.fori_loop(0, CH // SUB, chunk_body, 0)


@functools.partial(
    pl.kernel,
    out_type=jax.ShapeDtypeStruct((BINS,), jnp.float32),
    mesh=_mesh,
    scratch_types=[
        pltpu.VMEM((CHB,), jnp.int32),
        pltpu.VMEM((CROWS, 128), jnp.int32),
        pltpu.VMEM((128,), jnp.float32),
        pltpu.VMEM((ZB,), jnp.float32),
        pltpu.VMEM((A * OBC,), jnp.float32),
        pltpu.VMEM((OBC * A,), jnp.float32),
        pltpu.VMEM_SHARED((SLICE + TRASH,), jnp.float32),
        pltpu.SemaphoreType.DMA,
    ],
    compiler_params=pltpu.CompilerParams(needs_layout_passes=False),
)
def _hist(bins_hbm, out_hbm,
          binbuf, idxrows, ones, zeros, obin, obout, hist_sp, sem):
    c = lax.axis_index("c")
    s = lax.axis_index("s")

    def fill_ones(i, carry):
        ones[pl.ds(i * 16, 16)] = jnp.ones((16,), jnp.float32)
        return carry
    lax.fori_loop(0, 128 // 16, fill_ones, 0)

    def fill_zeros(i, carry):
        zeros[pl.ds(i * 16, 16)] = jnp.zeros((16,), jnp.float32)
        return carry
    lax.fori_loop(0, ZB // 16, fill_zeros, 0)

    iota = lax.iota(jnp.int32, 16)
    scan_base = s * PERS

    for p in range(P):
        cell_base = (p * NC + c) * C
        bin_base = (p * NC + c) * SLICE

        for k in range(CELLS_T * A // ZB):
            pltpu.sync_copy(zeros,
                            hist_sp.at[pl.ds(s * CELLS_T * A + k * ZB, ZB)])

        @pl.when(s == 0)
        def _():
            pltpu.sync_copy(zeros.at[pl.ds(0, TRASH)],
                            hist_sp.at[pl.ds(SLICE, TRASH)])

        plsc.subcore_barrier()

        def chunk_body(j, carry):
            off = scan_base + j * CHB
            pltpu.sync_copy(bins_hbm.at[pl.ds(off, CHB)], binbuf)

            def row_body(r, carry2):
                def vec_body(q, carry3):
                    o = r * 128 + q * 16
                    b = binbuf[pl.ds(o, 16)]
                    aa = lax.bitwise_and(b, A - 1)
                    cell = lax.shift_right_logical(b, 3)
                    dc = cell - cell_base
                    ok = (dc >= 0) & (dc < C)
                    loc = aa * C + dc
                    tr = SLICE + ((o + iota + s * 128) & (TRASH - 1))
                    idxrows[r, pl.ds(q * 16, 16)] = jnp.where(ok, loc, tr)
                    return carry3
                return lax.fori_loop(0, 8, vec_body, carry2)
            lax.fori_loop(0, CROWS, row_body, 0)

            def fire(r, carry2):
                pltpu.async_copy(ones, hist_sp.at[idxrows.at[r]], sem,
                                 add=True)
                return carry2
            lax.fori_loop(0, CROWS, fire, 0)

            def drain(r, carry2):
                pltpu.make_async_copy(ones, hist_sp.at[idxrows.at[0]],
                                      sem).wait()
                return carry2
            lax.fori_loop(0, CROWS, drain, 0)
            return carry
        lax.fori_loop(0, PERS // CHB, chunk_body, 0)

        plsc.subcore_barrier()

        for k in range(CELLS_T // OBC):
            cb = s * CELLS_T + k * OBC
            for a in range(A):
                pltpu.sync_copy(hist_sp.at[pl.ds(a * C + cb, OBC)],
                                obin.at[pl.ds(a * OBC, OBC)])

            def nvec(i, carry):
                base16 = i * 16
                vs = [obin[pl.ds(a * OBC + base16, 16)] for a in range(A)]
                tot = vs[0]
                for a in range(1, A):
                    tot = tot + vs[a]
                denom = jnp.maximum(tot, jnp.float32(1.0))
                recip = jnp.float32(1.0) / denom
                oidx = (base16 + iota) * A
                for a in range(A):
                    plsc.store_scatter(obout, [oidx + a], vs[a] * recip)
                return carry
            lax.fori_loop(0, OBC // 16, nvec, 0)

            pltpu.sync_copy(
                obout,
                out_hbm.at[pl.ds(bin_base + cb * A, OBC * A)])

        plsc.subcore_barrier()


def kernel(obs, acs, cost_matrix):
    obs_h = obs[:, 0]
    obs_w = obs[:, 1]
    table = cost_matrix.reshape(-1)
    cost, bins = _cost_bins(obs_h, obs_w, acs, table)
    policy = _hist(bins).reshape(H, W, A)
    return cost, policy


# speedup vs baseline: 8.8543x; 1.9942x over previous
import functools

import jax
import jax.numpy as jnp
from jax import lax
from jax.experimental import pallas as pl
from jax.experimental.pallas import tpu as pltpu
from jax.experimental.pallas import tpu_sc as plsc

H = 1024
W = 1024
A = 8
N = 1048576

NC = 2
NS = 16
NW = NC * NS

CH = N // NW
SUB = 4096
ROWS = SUB // 128

P = 4
BINS = H * W * A
SLICE = BINS // (NC * P)
C = SLICE // A
TRASH = 2048
PERS = N // NS
CHB = 8192
CROWS = CHB // 128
CELLS_T = C // NS
OBC = 2048
ZB = 8192

_mesh = plsc.VectorSubcoreMesh(core_axis_name="c", subcore_axis_name="s")


@functools.partial(
    pl.kernel,
    out_type=(
        jax.ShapeDtypeStruct((N,), jnp.float32),
        jax.ShapeDtypeStruct((N,), jnp.int32),
    ),
    mesh=_mesh,
    scratch_types=[
        pltpu.VMEM((SUB,), jnp.int32),
        pltpu.VMEM((SUB,), jnp.int32),
        pltpu.VMEM((SUB,), jnp.int32),
        pltpu.VMEM((ROWS, 128), jnp.int32),
        pltpu.VMEM((SUB,), jnp.int32),
        pltpu.VMEM((SUB,), jnp.float32),
        pltpu.VMEM_SHARED((H * W,), jnp.float32),
        pltpu.SemaphoreType.DMA,
    ],
)
def _cost_bins(obs_h, obs_w, acs, table, cost_out, bins_out,
               hbuf, wbuf, abuf, cellrows, binbuf, costbuf, table_sp, sem):
    c = lax.axis_index("c")
    s = lax.axis_index("s")
    wid = s * NC + c
    tchunk = (H * W) // NS
    pltpu.sync_copy(table.at[pl.ds(s * tchunk, tchunk)],
                    table_sp.at[pl.ds(s * tchunk, tchunk)])
    plsc.subcore_barrier()
    base = wid * CH

    def chunk_body(j, carry):
        off = base + j * SUB
        pltpu.sync_copy(obs_h.at[pl.ds(off, SUB)], hbuf)
        pltpu.sync_copy(obs_w.at[pl.ds(off, SUB)], wbuf)
        pltpu.sync_copy(acs.at[pl.ds(off, SUB)], abuf)

        def row_body(r, carry2):
            def vec_body(q, carry3):
                o = r * 128 + q * 16
                hh = hbuf[pl.ds(o, 16)]
                ww = wbuf[pl.ds(o, 16)]
                aa = abuf[pl.ds(o, 16)]
                cell = hh * W + ww
                cellrows[r, pl.ds(q * 16, 16)] = cell
                binbuf[pl.ds(o, 16)] = cell * A + aa
                return carry3
            return lax.fori_loop(0, 8, vec_body, carry2)
        lax.fori_loop(0, ROWS, row_body, 0)

        def fire(r, carry2):
            pltpu.async_copy(table_sp.at[cellrows.at[r]],
                             costbuf.at[pl.ds(r * 128, 128)], sem)
            return carry2
        lax.fori_loop(0, ROWS, fire, 0)

        def drain(r, carry2):
            pltpu.make_async_copy(table_sp.at[cellrows.at[0]],
                                  costbuf.at[pl.ds(0, 128)], sem).wait()
            return carry2
        lax.fori_loop(0, ROWS, drain, 0)

        pltpu.sync_copy(costbuf, cost_out.at[pl.ds(off, SUB)])
        pltpu.sync_copy(binbuf, bins_out.at[pl.ds(off, SUB)])
        return carry
    lax.fori_loop(0, CH // SUB, chunk_body, 0)


@functools.partial(
    pl.kernel,
    out_type=jax.ShapeDtypeStruct((BINS,), jnp.float32),
    mesh=_mesh,
    scratch_types=[
        pltpu.VMEM((CHB,), jnp.int32),
        pltpu.VMEM((CROWS, 128), jnp.int32),
        pltpu.VMEM((128,), jnp.float32),
        pltpu.VMEM((ZB,), jnp.float32),
        pltpu.VMEM((A * OBC,), jnp.float32),
        pltpu.VMEM((OBC * A,), jnp.float32),
        pltpu.VMEM_SHARED((SLICE + TRASH,), jnp.float32),
        pltpu.SemaphoreType.DMA,
    ],
    compiler_params=pltpu.CompilerParams(needs_layout_passes=False),
)
def _hist(bins_hbm, out_hbm,
          binbuf, idxrows, ones, zeros, obin, obout, hist_sp, sem):
    c = lax.axis_index("c")
    s = lax.axis_index("s")

    def fill_ones(i, carry):
        ones[pl.ds(i * 16, 16)] = jnp.ones((16,), jnp.float32)
        return carry
    lax.fori_loop(0, 128 // 16, fill_ones, 0)

    def fill_zeros(i, carry):
        zeros[pl.ds(i * 16, 16)] = jnp.zeros((16,), jnp.float32)
        return carry
    lax.fori_loop(0, ZB // 16, fill_zeros, 0)

    iota = lax.iota(jnp.int32, 16)
    scan_base = s * PERS

    for p in range(P):
        cell_base = (p * NC + c) * C
        bin_base = (p * NC + c) * SLICE

        for k in range(CELLS_T * A // ZB):
            pltpu.sync_copy(zeros,
                            hist_sp.at[pl.ds(s * CELLS_T * A + k * ZB, ZB)])

        @pl.when(s == 0)
        def _():
            pltpu.sync_copy(zeros.at[pl.ds(0, TRASH)],
                            hist_sp.at[pl.ds(SLICE, TRASH)])

        plsc.subcore_barrier()

        def chunk_body(j, carry):
            off = scan_base + j * CHB
            pltpu.sync_copy(bins_hbm.at[pl.ds(off, CHB)], binbuf)

            def row_body(r, carry2):
                def vec_body(q, carry3):
                    o = r * 128 + q * 16
                    b = binbuf[pl.ds(o, 16)]
                    aa = lax.bitwise_and(b, A - 1)
                    cell = lax.shift_right_logical(b, 3)
                    dc = cell - cell_base
                    ok = (dc >= 0) & (dc < C)
                    loc = aa * C + dc
                    tr = SLICE + ((o + iota + s * 128) & (TRASH - 1))
                    idxrows[r, pl.ds(q * 16, 16)] = jnp.where(ok, loc, tr)
                    return carry3
                return lax.fori_loop(0, 8, vec_body, carry2)
            lax.fori_loop(0, CROWS, row_body, 0)

            def fire(r, carry2):
                pltpu.async_copy(ones, hist_sp.at[idxrows.at[r]], sem,
                                 add=True)
                return carry2
            lax.fori_loop(0, CROWS, fire, 0)

            def drain(r, carry2):
                pltpu.make_async_copy(ones, hist_sp.at[idxrows.at[0]],
                                      sem).wait()
                return carry2
            lax.fori_loop(0, CROWS, drain, 0)
            return carry
        lax.fori_loop(0, PERS // CHB, chunk_body, 0)

        plsc.subcore_barrier()

        for k in range(CELLS_T // OBC):
            cb = s * CELLS_T + k * OBC
            for a in range(A):
                pltpu.sync_copy(hist_sp.at[pl.ds(a * C + cb, OBC)],
                                obin.at[pl.ds(a * OBC, OBC)])

            def nvec(i, carry):
                base16 = i * 16
                vs = [obin[pl.ds(a * OBC + base16, 16)] for a in range(A)]
                tot = vs[0]
                for a in range(1, A):
                    tot = tot + vs[a]
                denom = jnp.maximum(tot, jnp.float32(1.0))
                recip = jnp.float32(1.0) / denom
                cell = base16 + iota
                hrel = lax.shift_right_logical(cell, 10)
                ww = lax.bitwise_and(cell, W - 1)
                wb = lax.shift_right_logical(ww, 7)
                wi = lax.bitwise_and(ww, 127)
                posb = hrel * (A * W) + wb * (A * 128) + wi
                for a in range(A):
                    plsc.store_scatter(obout, [posb + a * 128], vs[a] * recip)
                return carry
            lax.fori_loop(0, OBC // 16, nvec, 0)

            pltpu.sync_copy(
                obout,
                out_hbm.at[pl.ds(bin_base + cb * A, OBC * A)])

        plsc.subcore_barrier()


def kernel(obs, acs, cost_matrix):
    obs_h = obs[:, 0]
    obs_w = obs[:, 1]
    table = cost_matrix.reshape(-1)
    cost, bins = _cost_bins(obs_h, obs_w, acs, table)
    flat = _hist(bins)
    policy = (flat.reshape(H, W // 128, A, 128)
              .transpose(0, 1, 3, 2)
              .reshape(H, W, A))
    return cost, policy


# speedup vs baseline: 13.6110x; 1.5372x over previous
import functools

import jax
import jax.numpy as jnp
from jax import lax
from jax.experimental import pallas as pl
from jax.experimental.pallas import tpu as pltpu
from jax.experimental.pallas import tpu_sc as plsc

H = 1024
W = 1024
A = 8
N = 1048576

NC = 2
NS = 16
NW = NC * NS

CH = N // NW
SUB = 4096
ROWS = SUB // 128

P = 4
BINS = H * W * A
SLICE = BINS // (NC * P)
C = SLICE // A
TRASH = 2048
PERS = N // NS
CHB = 8192
CROWS = CHB // 128
CELLS_T = C // NS
OBC = 1024
ZB = 2048

_mesh = plsc.VectorSubcoreMesh(core_axis_name="c", subcore_axis_name="s")


@functools.partial(
    pl.kernel,
    out_type=(
        jax.ShapeDtypeStruct((N,), jnp.float32),
        jax.ShapeDtypeStruct((N,), jnp.int32),
    ),
    mesh=_mesh,
    scratch_types=[
        pltpu.VMEM((SUB,), jnp.int32),
        pltpu.VMEM((SUB,), jnp.int32),
        pltpu.VMEM((SUB,), jnp.int32),
        pltpu.VMEM((ROWS, 128), jnp.int32),
        pltpu.VMEM((SUB,), jnp.int32),
        pltpu.VMEM((SUB,), jnp.float32),
        pltpu.VMEM_SHARED((H * W,), jnp.float32),
        pltpu.SemaphoreType.DMA,
    ],
)
def _cost_bins(obs_h, obs_w, acs, table, cost_out, bins_out,
               hbuf, wbuf, abuf, cellrows, binbuf, costbuf, table_sp, sem):
    c = lax.axis_index("c")
    s = lax.axis_index("s")
    wid = s * NC + c
    tchunk = (H * W) // NS
    pltpu.sync_copy(table.at[pl.ds(s * tchunk, tchunk)],
                    table_sp.at[pl.ds(s * tchunk, tchunk)])
    plsc.subcore_barrier()
    base = wid * CH

    def chunk_body(j, carry):
        off = base + j * SUB
        pltpu.sync_copy(obs_h.at[pl.ds(off, SUB)], hbuf)
        pltpu.sync_copy(obs_w.at[pl.ds(off, SUB)], wbuf)
        pltpu.sync_copy(acs.at[pl.ds(off, SUB)], abuf)

        def row_body(r, carry2):
            def vec_body(q, carry3):
                o = r * 128 + q * 16
                hh = hbuf[pl.ds(o, 16)]
                ww = wbuf[pl.ds(o, 16)]
                aa = abuf[pl.ds(o, 16)]
                cell = hh * W + ww
                cellrows[r, pl.ds(q * 16, 16)] = cell
                binbuf[pl.ds(o, 16)] = cell * A + aa
                return carry3
            return lax.fori_loop(0, 8, vec_body, carry2)
        lax.fori_loop(0, ROWS, row_body, 0)

        def fire(r, carry2):
            pltpu.async_copy(table_sp.at[cellrows.at[r]],
                             costbuf.at[pl.ds(r * 128, 128)], sem)
            return carry2
        lax.fori_loop(0, ROWS, fire, 0)

        def drain(r, carry2):
            pltpu.make_async_copy(table_sp.at[cellrows.at[0]],
                                  costbuf.at[pl.ds(0, 128)], sem).wait()
            return carry2
        lax.fori_loop(0, ROWS, drain, 0)

        pltpu.sync_copy(costbuf, cost_out.at[pl.ds(off, SUB)])
        pltpu.sync_copy(binbuf, bins_out.at[pl.ds(off, SUB)])
        return carry
    lax.fori_loop(0, CH // SUB, chunk_body, 0)


@functools.partial(
    pl.kernel,
    out_type=jax.ShapeDtypeStruct((BINS,), jnp.float32),
    mesh=_mesh,
    scratch_types=[
        pltpu.VMEM((CHB,), jnp.int32),
        pltpu.VMEM((CHB,), jnp.int32),
        pltpu.VMEM((CROWS, 128), jnp.int32),
        pltpu.VMEM((CROWS, 128), jnp.int32),
        pltpu.VMEM((128,), jnp.float32),
        pltpu.VMEM((ZB,), jnp.float32),
        pltpu.VMEM((A * OBC,), jnp.float32),
        pltpu.VMEM((OBC * A,), jnp.float32),
        pltpu.VMEM((OBC * A,), jnp.float32),
        pltpu.VMEM_SHARED((SLICE + TRASH,), jnp.float32),
        pltpu.SemaphoreType.DMA,
        pltpu.SemaphoreType.DMA,
        pltpu.SemaphoreType.DMA,
        pltpu.SemaphoreType.DMA,
    ],
    compiler_params=pltpu.CompilerParams(needs_layout_passes=False),
)
def _hist(bins_hbm, out_hbm,
          bbuf0, bbuf1, ibuf0, ibuf1, ones, zeros, obin, ob0, ob1,
          hist_sp, sem_stage, sem_scat, sem_wi, sem_wo):
    c = lax.axis_index("c")
    s = lax.axis_index("s")

    def fill_ones(i, carry):
        ones[pl.ds(i * 16, 16)] = jnp.ones((16,), jnp.float32)
        return carry
    lax.fori_loop(0, 128 // 16, fill_ones, 0)

    def fill_zeros(i, carry):
        zeros[pl.ds(i * 16, 16)] = jnp.zeros((16,), jnp.float32)
        return carry
    lax.fori_loop(0, ZB // 16, fill_zeros, 0)

    iota = lax.iota(jnp.int32, 16)
    tvec = jnp.int32(SLICE) + s * 16 + iota
    scan_base = s * PERS
    bbufs = (bbuf0, bbuf1)
    ibufs = (ibuf0, ibuf1)
    obufs = (ob0, ob1)
    NCH = PERS // CHB

    def scan_chunk(bbuf, ibuf, cell_base):
        def row_body(r, carry):
            def vec_body(q, carry3):
                o = r * 128 + q * 16
                b = bbuf[pl.ds(o, 16)]
                aa = lax.bitwise_and(b, A - 1)
                cell = lax.shift_right_logical(b, 3)
                dc = cell - cell_base
                ok = plsc.bitcast(dc, jnp.uint32) < jnp.uint32(C)
                loc = lax.shift_left(aa, 17) + dc
                ibuf[r, pl.ds(q * 16, 16)] = jnp.where(ok, loc, tvec)
                return carry3
            return lax.fori_loop(0, 8, vec_body, carry)
        lax.fori_loop(0, CROWS, row_body, 0)

    def fire_scatter(ibuf):
        def fire(r, carry):
            pltpu.async_copy(ones, hist_sp.at[ibuf.at[r]], sem_scat,
                             add=True)
            return carry
        lax.fori_loop(0, CROWS, fire, 0)

    def drain_scatter():
        def drain(r, carry):
            pltpu.make_async_copy(ones, hist_sp.at[ibuf0.at[0]],
                                  sem_scat).wait()
            return carry
        lax.fori_loop(0, CROWS, drain, 0)

    for p in range(P):
        cell_base = (p * NC + c) * C
        bin_base = (p * NC + c) * SLICE

        for k in range(CELLS_T * A // ZB):
            pltpu.sync_copy(zeros,
                            hist_sp.at[pl.ds(s * CELLS_T * A + k * ZB, ZB)])

        @pl.when(s == 0)
        def _():
            pltpu.sync_copy(zeros.at[pl.ds(0, TRASH)],
                            hist_sp.at[pl.ds(SLICE, TRASH)])

        plsc.subcore_barrier()

        pltpu.async_copy(bins_hbm.at[pl.ds(scan_base, CHB)], bbufs[0],
                         sem_stage)
        for j in range(NCH):
            pltpu.make_async_copy(bins_hbm.at[pl.ds(scan_base, CHB)],
                                  bbufs[j % 2], sem_stage).wait()
            if j + 1 < NCH:
                pltpu.async_copy(
                    bins_hbm.at[pl.ds(scan_base + (j + 1) * CHB, CHB)],
                    bbufs[(j + 1) % 2], sem_stage)
            scan_chunk(bbufs[j % 2], ibufs[j % 2], cell_base)
            if j > 0:
                drain_scatter()
            fire_scatter(ibufs[j % 2])
        drain_scatter()

        plsc.subcore_barrier()

        for k in range(CELLS_T // OBC):
            cb = s * CELLS_T + k * OBC
            for a in range(A):
                pltpu.async_copy(hist_sp.at[pl.ds(a * C + cb, OBC)],
                                 obin.at[pl.ds(a * OBC, OBC)], sem_wi)
            for a in range(A):
                pltpu.make_async_copy(hist_sp.at[pl.ds(0, OBC)],
                                      obin.at[pl.ds(0, OBC)], sem_wi).wait()
            if k >= 2:
                pltpu.make_async_copy(obufs[k % 2],
                                      out_hbm.at[pl.ds(bin_base, OBC * A)],
                                      sem_wo).wait()
            ob = obufs[k % 2]

            def nvec(i, carry):
                base16 = i * 16
                vs = [obin[pl.ds(a * OBC + base16, 16)] for a in range(A)]
                tot = vs[0]
                for a in range(1, A):
                    tot = tot + vs[a]
                denom = jnp.maximum(tot, jnp.float32(1.0))
                recip = jnp.float32(1.0) / denom
                cell = base16 + iota
                hrel = lax.shift_right_logical(cell, 10)
                ww = lax.bitwise_and(cell, W - 1)
                wb = lax.shift_right_logical(ww, 7)
                wi = lax.bitwise_and(ww, 127)
                posb = hrel * (A * W) + wb * (A * 128) + wi
                for a in range(A):
                    plsc.store_scatter(ob, [posb + a * 128], vs[a] * recip)
                return carry
            lax.fori_loop(0, OBC // 16, nvec, 0)

            pltpu.async_copy(ob, out_hbm.at[pl.ds(bin_base + cb * A, OBC * A)],
                             sem_wo)
        for _ in range(2):
            pltpu.make_async_copy(obufs[0],
                                  out_hbm.at[pl.ds(bin_base, OBC * A)],
                                  sem_wo).wait()

        plsc.subcore_barrier()


def kernel(obs, acs, cost_matrix):
    obs_h = obs[:, 0]
    obs_w = obs[:, 1]
    table = cost_matrix.reshape(-1)
    cost, bins = _cost_bins(obs_h, obs_w, acs, table)
    flat = _hist(bins)
    policy = (flat.reshape(H, W // 128, A, 128)
              .transpose(0, 1, 3, 2)
              .reshape(H, W, A))
    return cost, policy


# speedup vs baseline: 15.5399x; 1.1417x over previous
import functools

import jax
import jax.numpy as jnp
from jax import lax
from jax.experimental import pallas as pl
from jax.experimental.pallas import tpu as pltpu
from jax.experimental.pallas import tpu_sc as plsc

H = 1024
W = 1024
A = 8
N = 1048576

NC = 2
NS = 16
NW = NC * NS

CH = N // NW
SUB = 4096
ROWS = SUB // 128

P = 4
BINS = H * W * A
SLICE = BINS // (NC * P)
C = SLICE // A
TRASH = 2048
PERS = N // NS
CHB = 8192
CROWS = CHB // 128
CELLS_T = C // NS
OBC = 1024
ZB = 2048

_mesh = plsc.VectorSubcoreMesh(core_axis_name="c", subcore_axis_name="s")


@functools.partial(
    pl.kernel,
    out_type=(
        jax.ShapeDtypeStruct((N,), jnp.float32),
        jax.ShapeDtypeStruct((N,), jnp.int32),
    ),
    mesh=_mesh,
    scratch_types=[
        pltpu.VMEM((SUB,), jnp.int32),
        pltpu.VMEM((SUB,), jnp.int32),
        pltpu.VMEM((SUB,), jnp.int32),
        pltpu.VMEM((SUB,), jnp.int32),
        pltpu.VMEM((SUB,), jnp.int32),
        pltpu.VMEM((SUB,), jnp.int32),
        pltpu.VMEM((ROWS, 128), jnp.int32),
        pltpu.VMEM((ROWS, 128), jnp.int32),
        pltpu.VMEM((SUB,), jnp.int32),
        pltpu.VMEM((SUB,), jnp.int32),
        pltpu.VMEM((SUB,), jnp.float32),
        pltpu.VMEM((SUB,), jnp.float32),
        pltpu.VMEM_SHARED((H * W,), jnp.float32),
        pltpu.SemaphoreType.DMA,
        pltpu.SemaphoreType.DMA,
        pltpu.SemaphoreType.DMA,
    ],
)
def _cost_bins(obs_h, obs_w, acs, table, cost_out, bins_out,
               h0, h1, w0, w1, a0, a1, cr0, cr1, bb0, bb1, cb0, cb1,
               table_sp, sem_stage, sem_g, sem_w):
    c = lax.axis_index("c")
    s = lax.axis_index("s")
    wid = s * NC + c
    tchunk = (H * W) // NS
    pltpu.sync_copy(table.at[pl.ds(s * tchunk, tchunk)],
                    table_sp.at[pl.ds(s * tchunk, tchunk)])
    plsc.subcore_barrier()
    base = wid * CH
    hb = (h0, h1)
    wb = (w0, w1)
    ab = (a0, a1)
    crs = (cr0, cr1)
    bbs = (bb0, bb1)
    cbs = (cb0, cb1)
    NCHK = CH // SUB

    def stage(j):
        off = base + j * SUB
        pltpu.async_copy(obs_h.at[pl.ds(off, SUB)], hb[j % 2], sem_stage)
        pltpu.async_copy(obs_w.at[pl.ds(off, SUB)], wb[j % 2], sem_stage)
        pltpu.async_copy(acs.at[pl.ds(off, SUB)], ab[j % 2], sem_stage)

    def wait_stage(j):
        for _ in range(3):
            pltpu.make_async_copy(obs_h.at[pl.ds(base, SUB)], hb[j % 2],
                                  sem_stage).wait()

    def compute(j):
        hbuf, wbuf, abuf = hb[j % 2], wb[j % 2], ab[j % 2]
        cellrows, binbuf = crs[j % 2], bbs[j % 2]

        @plsc.parallel_loop(0, SUB // 16, unroll=4)
        def _(i):
            o = i * 16
            hh = hbuf[pl.ds(o, 16)]
            ww = wbuf[pl.ds(o, 16)]
            aa = abuf[pl.ds(o, 16)]
            cell = lax.shift_left(hh, 10) + ww
            r = lax.shift_right_logical(i, 3)
            q = lax.bitwise_and(i, 7)
            cellrows[r, pl.ds(q * 16, 16)] = cell
            binbuf[pl.ds(o, 16)] = lax.shift_left(cell, 3) + aa

    def fire_gathers(j):
        cellrows, costbuf = crs[j % 2], cbs[j % 2]

        def fire(r, carry):
            pltpu.async_copy(table_sp.at[cellrows.at[r]],
                             costbuf.at[pl.ds(r * 128, 128)], sem_g)
            return carry
        lax.fori_loop(0, ROWS, fire, 0)

    def drain_gathers():
        def drain(r, carry):
            pltpu.make_async_copy(table_sp.at[cr0.at[0]],
                                  cb0.at[pl.ds(0, 128)], sem_g).wait()
            return carry
        lax.fori_loop(0, ROWS, drain, 0)

    def fire_writes(j):
        off = base + j * SUB
        pltpu.async_copy(cbs[j % 2], cost_out.at[pl.ds(off, SUB)], sem_w)
        pltpu.async_copy(bbs[j % 2], bins_out.at[pl.ds(off, SUB)], sem_w)

    def drain_writes():
        pltpu.make_async_copy(cb0, cost_out.at[pl.ds(base, SUB)],
                              sem_w).wait()
        pltpu.make_async_copy(bb0, bins_out.at[pl.ds(base, SUB)],
                              sem_w).wait()

    stage(0)
    for j in range(NCHK):
        wait_stage(j)
        if j + 1 < NCHK:
            stage(j + 1)
        if j >= 2:
            drain_writes()
        compute(j)
        if j >= 1:
            drain_gathers()
            fire_writes(j - 1)
        fire_gathers(j)
    drain_gathers()
    fire_writes(NCHK - 1)
    drain_writes()
    drain_writes()


@functools.partial(
    pl.kernel,
    out_type=jax.ShapeDtypeStruct((BINS,), jnp.float32),
    mesh=_mesh,
    scratch_types=[
        pltpu.VMEM((CHB,), jnp.int32),
        pltpu.VMEM((CHB,), jnp.int32),
        pltpu.VMEM((CROWS, 128), jnp.int32),
        pltpu.VMEM((CROWS, 128), jnp.int32),
        pltpu.VMEM((128,), jnp.float32),
        pltpu.VMEM((ZB,), jnp.float32),
        pltpu.VMEM((A * OBC,), jnp.float32),
        pltpu.VMEM((OBC * A,), jnp.float32),
        pltpu.VMEM((OBC * A,), jnp.float32),
        pltpu.VMEM_SHARED((SLICE + TRASH,), jnp.float32),
        pltpu.SemaphoreType.DMA,
        pltpu.SemaphoreType.DMA,
        pltpu.SemaphoreType.DMA,
        pltpu.SemaphoreType.DMA,
    ],
    compiler_params=pltpu.CompilerParams(needs_layout_passes=False),
)
def _hist(bins_hbm, out_hbm,
          bbuf0, bbuf1, ibuf0, ibuf1, ones, zeros, obin, ob0, ob1,
          hist_sp, sem_stage, sem_scat, sem_wi, sem_wo):
    c = lax.axis_index("c")
    s = lax.axis_index("s")

    def fill_ones(i, carry):
        ones[pl.ds(i * 16, 16)] = jnp.ones((16,), jnp.float32)
        return carry
    lax.fori_loop(0, 128 // 16, fill_ones, 0)

    def fill_zeros(i, carry):
        zeros[pl.ds(i * 16, 16)] = jnp.zeros((16,), jnp.float32)
        return carry
    lax.fori_loop(0, ZB // 16, fill_zeros, 0)

    iota = lax.iota(jnp.int32, 16)
    tvec = jnp.int32(SLICE) + s * 16 + iota
    scan_base = s * PERS
    bbufs = (bbuf0, bbuf1)
    ibufs = (ibuf0, ibuf1)
    obufs = (ob0, ob1)
    NCH = PERS // CHB

    def scan_chunk(bbuf, ibuf, cell_base):
        @plsc.parallel_loop(0, CHB // 16, unroll=8)
        def _(i):
            o = i * 16
            b = bbuf[pl.ds(o, 16)]
            aa = lax.bitwise_and(b, A - 1)
            cell = lax.shift_right_logical(b, 3)
            dc = cell - cell_base
            ok = plsc.bitcast(dc, jnp.uint32) < jnp.uint32(C)
            loc = lax.shift_left(aa, 17) + dc
            r = lax.shift_right_logical(i, 3)
            q = lax.bitwise_and(i, 7)
            ibuf[r, pl.ds(q * 16, 16)] = jnp.where(ok, loc, tvec)

    def fire_scatter(ibuf):
        def fire(r, carry):
            pltpu.async_copy(ones, hist_sp.at[ibuf.at[r]], sem_scat,
                             add=True)
            return carry
        lax.fori_loop(0, CROWS, fire, 0)

    def drain_scatter():
        def drain(r, carry):
            pltpu.make_async_copy(ones, hist_sp.at[ibuf0.at[0]],
                                  sem_scat).wait()
            return carry
        lax.fori_loop(0, CROWS, drain, 0)

    for p in range(P):
        cell_base = (p * NC + c) * C
        bin_base = (p * NC + c) * SLICE

        for k in range(CELLS_T * A // ZB):
            pltpu.async_copy(zeros,
                             hist_sp.at[pl.ds(s * CELLS_T * A + k * ZB, ZB)],
                             sem_stage)

        @pl.when(s == 0)
        def _():
            pltpu.sync_copy(zeros.at[pl.ds(0, TRASH)],
                            hist_sp.at[pl.ds(SLICE, TRASH)])

        for k in range(CELLS_T * A // ZB):
            pltpu.make_async_copy(
                zeros, hist_sp.at[pl.ds(s * CELLS_T * A, ZB)],
                sem_stage).wait()

        plsc.subcore_barrier()

        pltpu.async_copy(bins_hbm.at[pl.ds(scan_base, CHB)], bbufs[0],
                         sem_stage)
        for j in range(NCH):
            pltpu.make_async_copy(bins_hbm.at[pl.ds(scan_base, CHB)],
                                  bbufs[j % 2], sem_stage).wait()
            if j + 1 < NCH:
                pltpu.async_copy(
                    bins_hbm.at[pl.ds(scan_base + (j + 1) * CHB, CHB)],
                    bbufs[(j + 1) % 2], sem_stage)
            scan_chunk(bbufs[j % 2], ibufs[j % 2], cell_base)
            if j > 0:
                drain_scatter()
            fire_scatter(ibufs[j % 2])
        drain_scatter()

        plsc.subcore_barrier()

        for k in range(CELLS_T // OBC):
            cb = s * CELLS_T + k * OBC
            for a in range(A):
                pltpu.async_copy(hist_sp.at[pl.ds(a * C + cb, OBC)],
                                 obin.at[pl.ds(a * OBC, OBC)], sem_wi)
            for a in range(A):
                pltpu.make_async_copy(hist_sp.at[pl.ds(0, OBC)],
                                      obin.at[pl.ds(0, OBC)], sem_wi).wait()
            if k >= 2:
                pltpu.make_async_copy(obufs[k % 2],
                                      out_hbm.at[pl.ds(bin_base, OBC * A)],
                                      sem_wo).wait()
            ob = obufs[k % 2]

            def nvec(i, carry):
                base16 = i * 16
                vs = [obin[pl.ds(a * OBC + base16, 16)] for a in range(A)]
                tot = vs[0]
                for a in range(1, A):
                    tot = tot + vs[a]
                denom = jnp.maximum(tot, jnp.float32(1.0))
                recip = jnp.float32(1.0) / denom
                cell = base16 + iota
                hrel = lax.shift_right_logical(cell, 10)
                ww = lax.bitwise_and(cell, W - 1)
                wb = lax.shift_right_logical(ww, 7)
                wi = lax.bitwise_and(ww, 127)
                posb = hrel * (A * W) + wb * (A * 128) + wi
                for a in range(A):
                    plsc.store_scatter(ob, [posb + a * 128], vs[a] * recip)
                return carry
            lax.fori_loop(0, OBC // 16, nvec, 0)

            pltpu.async_copy(ob, out_hbm.at[pl.ds(bin_base + cb * A, OBC * A)],
                             sem_wo)
        for _ in range(2):
            pltpu.make_async_copy(obufs[0],
                                  out_hbm.at[pl.ds(bin_base, OBC * A)],
                                  sem_wo).wait()

        plsc.subcore_barrier()


def kernel(obs, acs, cost_matrix):
    obs_h = obs[:, 0]
    obs_w = obs[:, 1]
    table = cost_matrix.reshape(-1)
    cost, bins = _cost_bins(obs_h, obs_w, acs, table)
    flat = _hist(bins)
    policy = (flat.reshape(H, W // 128, A, 128)
              .transpose(0, 1, 3, 2)
              .reshape(H, W, A))
    return cost, policy


# speedup vs baseline: 19.1500x; 1.2323x over previous
import functools

import jax
import jax.numpy as jnp
from jax import lax
from jax.experimental import pallas as pl
from jax.experimental.pallas import tpu as pltpu
from jax.experimental.pallas import tpu_sc as plsc

H = 1024
W = 1024
A = 8
N = 1048576

NC = 2
NS = 16
NW = NC * NS

CH = N // NW
SUB = 4096
ROWS = SUB // 128

P = 4
BINS = H * W * A
SLICE = BINS // (NC * P)
C = SLICE // A
TRASH = 2048
PERS = N // NS
CHB = 8192
CROWS = CHB // 128
CELLS_T = C // NS
OBC = 1024
ZB = 2048

_mesh = plsc.VectorSubcoreMesh(core_axis_name="c", subcore_axis_name="s")


@functools.partial(
    pl.kernel,
    out_type=(
        jax.ShapeDtypeStruct((N,), jnp.float32),
        jax.ShapeDtypeStruct((N,), jnp.int32),
    ),
    mesh=_mesh,
    scratch_types=[
        pltpu.VMEM((SUB,), jnp.int32),
        pltpu.VMEM((SUB,), jnp.int32),
        pltpu.VMEM((SUB,), jnp.int32),
        pltpu.VMEM((SUB,), jnp.int32),
        pltpu.VMEM((SUB,), jnp.int32),
        pltpu.VMEM((SUB,), jnp.int32),
        pltpu.VMEM((ROWS, 128), jnp.int32),
        pltpu.VMEM((ROWS, 128), jnp.int32),
        pltpu.VMEM((SUB,), jnp.int32),
        pltpu.VMEM((SUB,), jnp.int32),
        pltpu.VMEM((SUB,), jnp.float32),
        pltpu.VMEM((SUB,), jnp.float32),
        pltpu.VMEM_SHARED((H * W,), jnp.float32),
        pltpu.SemaphoreType.DMA,
        pltpu.SemaphoreType.DMA,
        pltpu.SemaphoreType.DMA,
    ],
)
def _cost_bins(obs_h, obs_w, acs, table, cost_out, bins_out,
               h0, h1, w0, w1, a0, a1, cr0, cr1, bb0, bb1, cb0, cb1,
               table_sp, sem_stage, sem_g, sem_w):
    c = lax.axis_index("c")
    s = lax.axis_index("s")
    wid = s * NC + c
    tchunk = (H * W) // NS
    pltpu.sync_copy(table.at[pl.ds(s * tchunk, tchunk)],
                    table_sp.at[pl.ds(s * tchunk, tchunk)])
    plsc.subcore_barrier()
    base = wid * CH
    hb = (h0, h1)
    wb = (w0, w1)
    ab = (a0, a1)
    crs = (cr0, cr1)
    bbs = (bb0, bb1)
    cbs = (cb0, cb1)
    NCHK = CH // SUB

    def stage(j):
        off = base + j * SUB
        pltpu.async_copy(obs_h.at[pl.ds(off, SUB)], hb[j % 2], sem_stage)
        pltpu.async_copy(obs_w.at[pl.ds(off, SUB)], wb[j % 2], sem_stage)
        pltpu.async_copy(acs.at[pl.ds(off, SUB)], ab[j % 2], sem_stage)

    def wait_stage(j):
        for _ in range(3):
            pltpu.make_async_copy(obs_h.at[pl.ds(base, SUB)], hb[j % 2],
                                  sem_stage).wait()

    def compute(j):
        hbuf, wbuf, abuf = hb[j % 2], wb[j % 2], ab[j % 2]
        cellrows, binbuf = crs[j % 2], bbs[j % 2]

        @plsc.parallel_loop(0, SUB // 16, unroll=4)
        def _(i):
            o = i * 16
            hh = hbuf[pl.ds(o, 16)]
            ww = wbuf[pl.ds(o, 16)]
            aa = abuf[pl.ds(o, 16)]
            cell = lax.shift_left(hh, 10) + ww
            r = lax.shift_right_logical(i, 3)
            q = lax.bitwise_and(i, 7)
            cellrows[r, pl.ds(q * 16, 16)] = cell
            binbuf[pl.ds(o, 16)] = lax.shift_left(cell, 3) + aa

    def fire_gathers(j):
        cellrows, costbuf = crs[j % 2], cbs[j % 2]

        def fire(r, carry):
            pltpu.async_copy(table_sp.at[cellrows.at[r]],
                             costbuf.at[pl.ds(r * 128, 128)], sem_g)
            return carry
        lax.fori_loop(0, ROWS, fire, 0)

    def drain_gathers():
        def drain(r, carry):
            pltpu.make_async_copy(table_sp.at[cr0.at[0]],
                                  cb0.at[pl.ds(0, 128)], sem_g).wait()
            return carry
        lax.fori_loop(0, ROWS, drain, 0)

    def fire_writes(j):
        off = base + j * SUB
        pltpu.async_copy(cbs[j % 2], cost_out.at[pl.ds(off, SUB)], sem_w)
        pltpu.async_copy(bbs[j % 2], bins_out.at[pl.ds(off, SUB)], sem_w)

    def drain_writes():
        pltpu.make_async_copy(cb0, cost_out.at[pl.ds(base, SUB)],
                              sem_w).wait()
        pltpu.make_async_copy(bb0, bins_out.at[pl.ds(base, SUB)],
                              sem_w).wait()

    stage(0)
    for j in range(NCHK):
        wait_stage(j)
        if j + 1 < NCHK:
            stage(j + 1)
        if j >= 2:
            drain_writes()
        compute(j)
        if j >= 1:
            drain_gathers()
            fire_writes(j - 1)
        fire_gathers(j)
    drain_gathers()
    fire_writes(NCHK - 1)
    drain_writes()
    drain_writes()


@functools.partial(
    pl.kernel,
    out_type=jax.ShapeDtypeStruct((BINS,), jnp.float32),
    mesh=_mesh,
    scratch_types=[
        pltpu.VMEM((CHB,), jnp.int32),
        pltpu.VMEM((CHB,), jnp.int32),
        pltpu.VMEM((CHB // 128 + 1, 128), jnp.int32),
        pltpu.VMEM((CHB // 128 + 1, 128), jnp.int32),
        pltpu.VMEM((128,), jnp.float32),
        pltpu.VMEM((ZB,), jnp.float32),
        pltpu.VMEM((A * OBC,), jnp.float32),
        pltpu.VMEM((OBC * A,), jnp.float32),
        pltpu.VMEM((OBC * A,), jnp.float32),
        pltpu.VMEM_SHARED((SLICE + TRASH,), jnp.float32),
        pltpu.SemaphoreType.DMA,
        pltpu.SemaphoreType.DMA,
        pltpu.SemaphoreType.DMA,
        pltpu.SemaphoreType.DMA,
    ],
    compiler_params=pltpu.CompilerParams(needs_layout_passes=False),
)
def _hist(bins_hbm, out_hbm,
          bbuf0, bbuf1, ibuf0, ibuf1, ones, zeros, obin, ob0, ob1,
          hist_sp, sem_stage, sem_scat, sem_wi, sem_wo):
    c = lax.axis_index("c")
    s = lax.axis_index("s")

    def fill_ones(i, carry):
        ones[pl.ds(i * 16, 16)] = jnp.ones((16,), jnp.float32)
        return carry
    lax.fori_loop(0, 128 // 16, fill_ones, 0)

    def fill_zeros(i, carry):
        zeros[pl.ds(i * 16, 16)] = jnp.zeros((16,), jnp.float32)
        return carry
    lax.fori_loop(0, ZB // 16, fill_zeros, 0)

    iota = lax.iota(jnp.int32, 16)
    tvec = jnp.int32(SLICE) + s * 16 + iota
    scan_base = s * PERS
    bbufs = (bbuf0, bbuf1)
    ibufs = (ibuf0, ibuf1)
    obufs = (ob0, ob1)
    NCH = PERS // CHB

    def scan_chunk(bbuf, ibuf, cell_base):
        @plsc.parallel_loop(0, CHB // 16, unroll=4,
                            carry=jnp.zeros((16,), jnp.int32))
        def cntv(i, cnt):
            o = i * 16
            b = bbuf[pl.ds(o, 16)]
            aa = lax.bitwise_and(b, A - 1)
            cell = lax.shift_right_logical(b, 3)
            dc = cell - cell_base
            ok = plsc.bitcast(dc, jnp.uint32) < jnp.uint32(C)
            loc = lax.shift_left(aa, 17) + dc
            oki = ok.astype(jnp.int32)
            cs = plsc.cumsum(oki)
            pos = cnt + cs - oki
            plsc.store_scatter(ibuf,
                               [lax.shift_right_logical(pos, 7),
                                lax.bitwise_and(pos, 127)], loc, mask=ok)
            return cnt + plsc.all_reduce_population_count(ok)
        cnt_s = lax.reduce_max(cntv, (0,))
        for k in range(8):
            pp = cnt_s + k * 16 + iota
            plsc.store_scatter(ibuf,
                               [lax.shift_right_logical(pp, 7),
                                lax.bitwise_and(pp, 127)], tvec)
        return lax.shift_right_logical(cnt_s + 127, 7)

    def fire_scatter(ibuf, nrows):
        def fire(r, carry):
            pltpu.async_copy(ones, hist_sp.at[ibuf.at[r]], sem_scat,
                             add=True)
            return carry
        lax.fori_loop(0, nrows, fire, 0)

    def drain_scatter(nrows):
        def drain(r, carry):
            pltpu.make_async_copy(ones, hist_sp.at[ibuf0.at[0]],
                                  sem_scat).wait()
            return carry
        lax.fori_loop(0, nrows, drain, 0)

    def pass_body(p, carry_p):
        cell_base = (p * NC + c) * C
        bin_base = (p * NC + c) * SLICE

        for k in range(CELLS_T * A // ZB):
            pltpu.async_copy(zeros,
                             hist_sp.at[pl.ds(s * CELLS_T * A + k * ZB, ZB)],
                             sem_stage)

        @pl.when(s == 0)
        def _():
            pltpu.sync_copy(zeros.at[pl.ds(0, TRASH)],
                            hist_sp.at[pl.ds(SLICE, TRASH)])

        for k in range(CELLS_T * A // ZB):
            pltpu.make_async_copy(
                zeros, hist_sp.at[pl.ds(s * CELLS_T * A, ZB)],
                sem_stage).wait()

        plsc.subcore_barrier()

        pltpu.async_copy(bins_hbm.at[pl.ds(scan_base, CHB)], bbufs[0],
                         sem_stage)
        nrows_prev = None
        for j in range(NCH):
            pltpu.make_async_copy(bins_hbm.at[pl.ds(scan_base, CHB)],
                                  bbufs[j % 2], sem_stage).wait()
            if j + 1 < NCH:
                pltpu.async_copy(
                    bins_hbm.at[pl.ds(scan_base + (j + 1) * CHB, CHB)],
                    bbufs[(j + 1) % 2], sem_stage)
            nrows_j = scan_chunk(bbufs[j % 2], ibufs[j % 2], cell_base)
            if nrows_prev is not None:
                drain_scatter(nrows_prev)
            fire_scatter(ibufs[j % 2], nrows_j)
            nrows_prev = nrows_j
        drain_scatter(nrows_prev)

        plsc.subcore_barrier()

        for k in range(CELLS_T // OBC):
            cb = s * CELLS_T + k * OBC
            for a in range(A):
                pltpu.async_copy(hist_sp.at[pl.ds(a * C + cb, OBC)],
                                 obin.at[pl.ds(a * OBC, OBC)], sem_wi)
            for a in range(A):
                pltpu.make_async_copy(hist_sp.at[pl.ds(0, OBC)],
                                      obin.at[pl.ds(0, OBC)], sem_wi).wait()
            if k >= 2:
                pltpu.make_async_copy(obufs[k % 2],
                                      out_hbm.at[pl.ds(bin_base, OBC * A)],
                                      sem_wo).wait()
            ob = obufs[k % 2]

            def nvec(i, carry):
                base16 = i * 16
                vs = [obin[pl.ds(a * OBC + base16, 16)] for a in range(A)]
                tot = vs[0]
                for a in range(1, A):
                    tot = tot + vs[a]
                denom = jnp.maximum(tot, jnp.float32(1.0))
                recip = jnp.float32(1.0) / denom
                cell = base16 + iota
                hrel = lax.shift_right_logical(cell, 10)
                ww = lax.bitwise_and(cell, W - 1)
                wb = lax.shift_right_logical(ww, 7)
                wi = lax.bitwise_and(ww, 127)
                posb = hrel * (A * W) + wb * (A * 128) + wi
                for a in range(A):
                    plsc.store_scatter(ob, [posb + a * 128], vs[a] * recip)
                return carry
            lax.fori_loop(0, OBC // 16, nvec, 0)

            pltpu.async_copy(ob, out_hbm.at[pl.ds(bin_base + cb * A, OBC * A)],
                             sem_wo)
        for _ in range(2):
            pltpu.make_async_copy(obufs[0],
                                  out_hbm.at[pl.ds(bin_base, OBC * A)],
                                  sem_wo).wait()

        plsc.subcore_barrier()
        return carry_p
    lax.fori_loop(0, P, pass_body, 0)


def kernel(obs, acs, cost_matrix):
    obs_h = obs[:, 0]
    obs_w = obs[:, 1]
    table = cost_matrix.reshape(-1)
    cost, bins = _cost_bins(obs_h, obs_w, acs, table)
    flat = _hist(bins)
    policy = (flat.reshape(H, W // 128, A, 128)
              .transpose(0, 1, 3, 2)
              .reshape(H, W, A))
    return cost, policy


# speedup vs baseline: 24.6411x; 1.2867x over previous
import functools

import jax
import jax.numpy as jnp
from jax import lax
from jax.experimental import pallas as pl
from jax.experimental.pallas import tpu as pltpu
from jax.experimental.pallas import tpu_sc as plsc

H = 1024
W = 1024
A = 8
N = 1048576

NC = 2
NS = 16
NW = NC * NS

CH = N // NW
SUB = 4096
ROWS = SUB // 128

P = 4
BINS = H * W * A
SLICE = BINS // (NC * P)
C = SLICE // A
TRASH = 2048
PERS = N // NS
CHB = 8192
CROWS = CHB // 128
CELLS_T = C // NS
OBC = 1024
ZB = 2048

_mesh = plsc.VectorSubcoreMesh(core_axis_name="c", subcore_axis_name="s")


@functools.partial(
    pl.kernel,
    out_type=(
        jax.ShapeDtypeStruct((N,), jnp.float32),
        jax.ShapeDtypeStruct((N,), jnp.int32),
    ),
    mesh=_mesh,
    scratch_types=[
        pltpu.VMEM((SUB,), jnp.int32),
        pltpu.VMEM((SUB,), jnp.int32),
        pltpu.VMEM((SUB,), jnp.int32),
        pltpu.VMEM((SUB,), jnp.int32),
        pltpu.VMEM((SUB,), jnp.int32),
        pltpu.VMEM((SUB,), jnp.int32),
        pltpu.VMEM((ROWS, 128), jnp.int32),
        pltpu.VMEM((ROWS, 128), jnp.int32),
        pltpu.VMEM((SUB,), jnp.int32),
        pltpu.VMEM((SUB,), jnp.int32),
        pltpu.VMEM((SUB,), jnp.float32),
        pltpu.VMEM((SUB,), jnp.float32),
        pltpu.VMEM_SHARED((H * W,), jnp.float32),
        pltpu.SemaphoreType.DMA,
        pltpu.SemaphoreType.DMA,
        pltpu.SemaphoreType.DMA,
    ],
)
def _cost_bins(obs_t, acs, table, cost_out, bins_out,
               h0, h1, w0, w1, a0, a1, cr0, cr1, bb0, bb1, cb0, cb1,
               table_sp, sem_stage, sem_g, sem_w):
    c = lax.axis_index("c")
    s = lax.axis_index("s")
    wid = s * NC + c
    tchunk = (H * W) // NS
    pltpu.sync_copy(table.at[pl.ds(s * tchunk, tchunk)],
                    table_sp.at[pl.ds(s * tchunk, tchunk)])
    plsc.subcore_barrier()
    base = wid * CH
    hb = (h0, h1)
    wb = (w0, w1)
    ab = (a0, a1)
    crs = (cr0, cr1)
    bbs = (bb0, bb1)
    cbs = (cb0, cb1)
    NCHK = CH // SUB

    def stage(j):
        off = base + j * SUB
        pltpu.async_copy(obs_t.at[0, pl.ds(off, SUB)], hb[j % 2], sem_stage)
        pltpu.async_copy(obs_t.at[1, pl.ds(off, SUB)], wb[j % 2], sem_stage)
        pltpu.async_copy(acs.at[pl.ds(off, SUB)], ab[j % 2], sem_stage)

    def wait_stage(j):
        for _ in range(3):
            pltpu.make_async_copy(obs_t.at[0, pl.ds(base, SUB)], hb[j % 2],
                                  sem_stage).wait()

    def compute(j):
        hbuf, wbuf, abuf = hb[j % 2], wb[j % 2], ab[j % 2]
        cellrows, binbuf = crs[j % 2], bbs[j % 2]

        @plsc.parallel_loop(0, SUB // 16, unroll=4)
        def _(i):
            o = i * 16
            hh = hbuf[pl.ds(o, 16)]
            ww = wbuf[pl.ds(o, 16)]
            aa = abuf[pl.ds(o, 16)]
            cell = lax.shift_left(hh, 10) + ww
            r = lax.shift_right_logical(i, 3)
            q = lax.bitwise_and(i, 7)
            cellrows[r, pl.ds(q * 16, 16)] = cell
            binbuf[pl.ds(o, 16)] = lax.shift_left(cell, 3) + aa

    def fire_gathers(j):
        cellrows, costbuf = crs[j % 2], cbs[j % 2]

        def fire(r, carry):
            pltpu.async_copy(table_sp.at[cellrows.at[r]],
                             costbuf.at[pl.ds(r * 128, 128)], sem_g)
            return carry
        lax.fori_loop(0, ROWS, fire, 0)

    def drain_gathers():
        def drain(r, carry):
            pltpu.make_async_copy(table_sp.at[cr0.at[0]],
                                  cb0.at[pl.ds(0, 128)], sem_g).wait()
            return carry
        lax.fori_loop(0, ROWS, drain, 0)

    def fire_writes(j):
        off = base + j * SUB
        pltpu.async_copy(cbs[j % 2], cost_out.at[pl.ds(off, SUB)], sem_w)
        pltpu.async_copy(bbs[j % 2], bins_out.at[pl.ds(off, SUB)], sem_w)

    def drain_writes():
        pltpu.make_async_copy(cb0, cost_out.at[pl.ds(base, SUB)],
                              sem_w).wait()
        pltpu.make_async_copy(bb0, bins_out.at[pl.ds(base, SUB)],
                              sem_w).wait()

    stage(0)
    for j in range(NCHK):
        wait_stage(j)
        if j + 1 < NCHK:
            stage(j + 1)
        if j >= 2:
            drain_writes()
        compute(j)
        if j >= 1:
            drain_gathers()
            fire_writes(j - 1)
        fire_gathers(j)
    drain_gathers()
    fire_writes(NCHK - 1)
    drain_writes()
    drain_writes()


@functools.partial(
    pl.kernel,
    out_type=jax.ShapeDtypeStruct((BINS,), jnp.float32),
    mesh=_mesh,
    scratch_types=[
        pltpu.VMEM((CHB,), jnp.int32),
        pltpu.VMEM((CHB,), jnp.int32),
        pltpu.VMEM((CHB // 128 + 1, 128), jnp.int32),
        pltpu.VMEM((CHB // 128 + 1, 128), jnp.int32),
        pltpu.VMEM((128,), jnp.float32),
        pltpu.VMEM((ZB,), jnp.float32),
        pltpu.VMEM((A * OBC,), jnp.float32),
        pltpu.VMEM((OBC * A,), jnp.float32),
        pltpu.VMEM((OBC * A,), jnp.float32),
        pltpu.VMEM_SHARED((SLICE + TRASH,), jnp.float32),
        pltpu.SemaphoreType.DMA,
        pltpu.SemaphoreType.DMA,
        pltpu.SemaphoreType.DMA,
        pltpu.SemaphoreType.DMA,
    ],
    compiler_params=pltpu.CompilerParams(needs_layout_passes=False),
)
def _hist(bins_hbm, out_hbm,
          bbuf0, bbuf1, ibuf0, ibuf1, ones, zeros, obin, ob0, ob1,
          hist_sp, sem_stage, sem_scat, sem_wi, sem_wo):
    c = lax.axis_index("c")
    s = lax.axis_index("s")

    def fill_ones(i, carry):
        ones[pl.ds(i * 16, 16)] = jnp.ones((16,), jnp.float32)
        return carry
    lax.fori_loop(0, 128 // 16, fill_ones, 0)

    def fill_zeros(i, carry):
        zeros[pl.ds(i * 16, 16)] = jnp.zeros((16,), jnp.float32)
        return carry
    lax.fori_loop(0, ZB // 16, fill_zeros, 0)

    iota = lax.iota(jnp.int32, 16)
    tvec = jnp.int32(SLICE) + s * 16 + iota
    scan_base = s * PERS
    bbufs = (bbuf0, bbuf1)
    ibufs = (ibuf0, ibuf1)
    obufs = (ob0, ob1)
    NCH = PERS // CHB

    def scan_chunk(bbuf, ibuf, cell_base):
        @plsc.parallel_loop(0, CHB // 16, unroll=4,
                            carry=jnp.zeros((16,), jnp.int32))
        def cntv(i, cnt):
            o = i * 16
            b = bbuf[pl.ds(o, 16)]
            aa = lax.bitwise_and(b, A - 1)
            cell = lax.shift_right_logical(b, 3)
            dc = cell - cell_base
            ok = plsc.bitcast(dc, jnp.uint32) < jnp.uint32(C)
            loc = lax.shift_left(aa, 17) + dc
            oki = ok.astype(jnp.int32)
            cs = plsc.cumsum(oki)
            pos = cnt + cs - oki
            plsc.store_scatter(ibuf,
                               [lax.shift_right_logical(pos, 7),
                                lax.bitwise_and(pos, 127)], loc, mask=ok)
            return cnt + plsc.all_reduce_population_count(ok)
        cnt_s = lax.reduce_max(cntv, (0,))
        for k in range(8):
            pp = cnt_s + k * 16 + iota
            plsc.store_scatter(ibuf,
                               [lax.shift_right_logical(pp, 7),
                                lax.bitwise_and(pp, 127)], tvec)
        return lax.shift_right_logical(cnt_s + 127, 7)

    def fire_scatter(ibuf, nrows):
        def fire(r, carry):
            pltpu.async_copy(ones, hist_sp.at[ibuf.at[r]], sem_scat,
                             add=True)
            return carry
        lax.fori_loop(0, nrows, fire, 0)

    def drain_scatter(nrows):
        def drain(r, carry):
            pltpu.make_async_copy(ones, hist_sp.at[ibuf0.at[0]],
                                  sem_scat).wait()
            return carry
        lax.fori_loop(0, nrows, drain, 0)

    def pass_body(p, carry_p):
        cell_base = (p * NC + c) * C
        bin_base = (p * NC + c) * SLICE

        for k in range(CELLS_T * A // ZB):
            pltpu.async_copy(zeros,
                             hist_sp.at[pl.ds(s * CELLS_T * A + k * ZB, ZB)],
                             sem_stage)

        @pl.when(s == 0)
        def _():
            pltpu.sync_copy(zeros.at[pl.ds(0, TRASH)],
                            hist_sp.at[pl.ds(SLICE, TRASH)])

        for k in range(CELLS_T * A // ZB):
            pltpu.make_async_copy(
                zeros, hist_sp.at[pl.ds(s * CELLS_T * A, ZB)],
                sem_stage).wait()

        plsc.subcore_barrier()

        pltpu.async_copy(bins_hbm.at[pl.ds(scan_base, CHB)], bbufs[0],
                         sem_stage)
        nrows_prev = None
        for j in range(NCH):
            pltpu.make_async_copy(bins_hbm.at[pl.ds(scan_base, CHB)],
                                  bbufs[j % 2], sem_stage).wait()
            if j + 1 < NCH:
                pltpu.async_copy(
                    bins_hbm.at[pl.ds(scan_base + (j + 1) * CHB, CHB)],
                    bbufs[(j + 1) % 2], sem_stage)
            nrows_j = scan_chunk(bbufs[j % 2], ibufs[j % 2], cell_base)
            if nrows_prev is not None:
                drain_scatter(nrows_prev)
            fire_scatter(ibufs[j % 2], nrows_j)
            nrows_prev = nrows_j
        drain_scatter(nrows_prev)

        plsc.subcore_barrier()

        for k in range(CELLS_T // OBC):
            cb = s * CELLS_T + k * OBC
            for a in range(A):
                pltpu.async_copy(hist_sp.at[pl.ds(a * C + cb, OBC)],
                                 obin.at[pl.ds(a * OBC, OBC)], sem_wi)
            for a in range(A):
                pltpu.make_async_copy(hist_sp.at[pl.ds(0, OBC)],
                                      obin.at[pl.ds(0, OBC)], sem_wi).wait()
            if k >= 2:
                pltpu.make_async_copy(obufs[k % 2],
                                      out_hbm.at[pl.ds(bin_base, OBC * A)],
                                      sem_wo).wait()
            ob = obufs[k % 2]

            @plsc.parallel_loop(0, OBC // 16, unroll=2)
            def nvec(i):
                base16 = i * 16
                vs = [obin[pl.ds(a * OBC + base16, 16)] for a in range(A)]
                tot = vs[0]
                for a in range(1, A):
                    tot = tot + vs[a]
                denom = jnp.maximum(tot, jnp.float32(1.0))
                recip = jnp.float32(1.0) / denom
                cell = base16 + iota
                hrel = lax.shift_right_logical(cell, 10)
                ww = lax.bitwise_and(cell, W - 1)
                wb = lax.shift_right_logical(ww, 7)
                wi = lax.bitwise_and(ww, 127)
                posb = hrel * (A * W) + wb * (A * 128) + wi
                for a in range(A):
                    plsc.store_scatter(ob, [posb + a * 128], vs[a] * recip)

            pltpu.async_copy(ob, out_hbm.at[pl.ds(bin_base + cb * A, OBC * A)],
                             sem_wo)
        for _ in range(2):
            pltpu.make_async_copy(obufs[0],
                                  out_hbm.at[pl.ds(bin_base, OBC * A)],
                                  sem_wo).wait()

        plsc.subcore_barrier()
        return carry_p
    lax.fori_loop(0, P, pass_body, 0)


def kernel(obs, acs, cost_matrix):
    table = cost_matrix.reshape(-1)
    cost, bins = _cost_bins(obs.T, acs, table)
    flat = _hist(bins)
    policy = (flat.reshape(H, W // 128, A, 128)
              .transpose(0, 1, 3, 2)
              .reshape(H, W, A))
    return cost, policy
